# baseline (speedup 1.0000x reference)
import functools

import jax
import jax.numpy as jnp
from jax import lax
from jax.experimental import pallas as pl
from jax.experimental.pallas import tpu as pltpu
from jax.experimental.pallas import tpu_sc as plsc

N = 10000
E = 320000
D = 128
ED = 16
G = 64
MD = 17
L = 3

NC = 2
NS = 16
NW = NC * NS
CHUNK = 64
NCH_A = 200
NCH_B = 120
EPS_A = NCH_A * CHUNK
EPS_B = NCH_B * CHUNK
EPAD = NS * (EPS_A + EPS_B)
NPAD = 10240
RPS = NPAD // NS
EB = 2048
LANES = 16


def _inproj(x, W, b):
    def body(x_ref, w_ref, b_ref, o_ref):
        o_ref[...] = jnp.dot(x_ref[...], w_ref[...],
                             preferred_element_type=jnp.float32) + b_ref[...]
    return pl.pallas_call(
        body, out_shape=jax.ShapeDtypeStruct((N, D), jnp.float32),
    )(x, W, b.reshape(1, D))


def _edge_mlp(ea, W1, b1, W2, b2):
    def body(ea_ref, w1_ref, b1_ref, w2_ref, b2_ref, o_ref):
        t = jnp.maximum(
            jnp.dot(ea_ref[...], w1_ref[...],
                    preferred_element_type=jnp.float32) + b1_ref[...], 0.0)
        o_ref[...] = jnp.dot(t, w2_ref[...],
                             preferred_element_type=jnp.float32) + b2_ref[...]
    return pl.pallas_call(
        body,
        grid=(EPAD // EB,),
        in_specs=[
            pl.BlockSpec((EB, ED), lambda i: (i, 0)),
            pl.BlockSpec((ED, D), lambda i: (0, 0)),
            pl.BlockSpec((1, D), lambda i: (0, 0)),
            pl.BlockSpec((D, D), lambda i: (0, 0)),
            pl.BlockSpec((1, D), lambda i: (0, 0)),
        ],
        out_specs=pl.BlockSpec((EB, D), lambda i: (i, 0)),
        out_shape=jax.ShapeDtypeStruct((EPAD, D), jnp.float32),
    )(ea, W1, b1.reshape(1, D), W2, b2.reshape(1, D))


def _bn_update(h, agg, g, be):
    def body(h_ref, a_ref, g_ref, be_ref, o_ref):
        z = h_ref[...] + a_ref[0] + a_ref[1]
        mean = jnp.mean(z, axis=0, keepdims=True)
        zc = z - mean
        var = jnp.mean(zc * zc, axis=0, keepdims=True)
        y = g_ref[...] * zc * lax.rsqrt(var + 1e-5) + be_ref[...]
        o_ref[...] = jnp.maximum(y, 0.0)
    return pl.pallas_call(
        body,
        grid=(1,),
        in_specs=[
            pl.BlockSpec((N, D), lambda i: (0, 0)),
            pl.BlockSpec((NC, N, D), lambda i: (0, 0, 0)),
            pl.BlockSpec((1, D), lambda i: (0, 0)),
            pl.BlockSpec((1, D), lambda i: (0, 0)),
        ],
        out_specs=pl.BlockSpec((N, D), lambda i: (0, 0)),
        out_shape=jax.ShapeDtypeStruct((N, D), jnp.float32),
    )(h, agg, g.reshape(1, D), be.reshape(1, D))


def _pool_fc(h, batch2d, mol_desc, wh, wm, bf):
    def body(h_ref, b_ref, md_ref, wh_ref, wm_ref, bf_ref, o_ref):
        gids = lax.broadcasted_iota(jnp.int32, (G, N), 0)
        M = (b_ref[...] == gids).astype(jnp.float32)
        sums = jnp.dot(M, h_ref[...], preferred_element_type=jnp.float32)
        counts = jnp.sum(M, axis=1, keepdims=True)
        pooled = sums / jnp.maximum(counts, 1.0)
        o_ref[...] = (jnp.sum(pooled * wh_ref[...], axis=1, keepdims=True)
                      + jnp.sum(md_ref[...] * wm_ref[...], axis=1, keepdims=True)
                      + bf_ref[0, 0])
    return pl.pallas_call(
        body, out_shape=jax.ShapeDtypeStruct((G, 1), jnp.float32),
    )(h, batch2d, mol_desc, wh, wm, bf.reshape(1, 1))


def _sc_messages(h, src2, dst2, he):
    mesh = plsc.VectorSubcoreMesh(core_axis_name="c", subcore_axis_name="s",
                                  num_cores=2)

    @functools.partial(
        pl.kernel,
        out_type=jax.ShapeDtypeStruct((NC, NPAD, D), jnp.float32),
        mesh=mesh,
        scratch_types=[
            pltpu.VMEM((4, CHUNK), jnp.int32),
            pltpu.VMEM((4, CHUNK), jnp.int32),
            pltpu.VMEM((2, CHUNK, D), jnp.float32),
            pltpu.VMEM((2, CHUNK, D), jnp.float32),
            pltpu.VMEM_SHARED((NPAD, D), jnp.float32),
            pltpu.SemaphoreType.DMA,
            pltpu.SemaphoreType.DMA,
            pltpu.SemaphoreType.DMA,
            pltpu.SemaphoreType.DMA,
            pltpu.SemaphoreType.DMA,
            pltpu.SemaphoreType.DMA,
            pltpu.SemaphoreType.DMA,
            pltpu.SemaphoreType.DMA,
            pltpu.SemaphoreType.DMA,
            pltpu.SemaphoreType.DMA,
        ],
    )
    def k(h_hbm, src_hbm, dst_hbm, he_hbm, out_hbm,
          src_v, dst_v, rows_v, he_v, agg_sh,
          is0, is1, is2, is3, gs0, gs1, hs0, hs1, ss0, ss1):
        cid = lax.axis_index("c")
        sid = lax.axis_index("s")
        on_a = cid == 0
        nch = jnp.where(on_a, NCH_A, NCH_B)
        ebase = jnp.where(on_a, sid * EPS_A, NS * EPS_A + sid * EPS_B)
        isems = (is0, is1, is2, is3)
        gsems = (gs0, gs1)
        hsems = (hs0, hs1)
        ssems = (ss0, ss1)

        zbuf = rows_v.at[0]

        @pl.loop(0, CHUNK)
        def _(r):
            zrow = zbuf.at[r]
            for v in range(D // LANES):
                zrow[pl.ds(v * LANES, LANES)] = jnp.zeros((LANES,),
                                                          jnp.float32)

        @pl.loop(0, RPS // CHUNK)
        def _(t):
            pltpu.sync_copy(
                zbuf, agg_sh.at[pl.ds(sid * RPS + t * CHUNK, CHUNK)])
        plsc.subcore_barrier()

        def start_idx(c, q):
            pltpu.async_copy(src_hbm.at[pl.ds(ebase + c * CHUNK, CHUNK)],
                             src_v.at[q], isems[q])
            pltpu.async_copy(dst_hbm.at[pl.ds(ebase + c * CHUNK, CHUNK)],
                             dst_v.at[q], isems[q])

        def wait_idx(c, q):
            pltpu.make_async_copy(
                src_hbm.at[pl.ds(ebase + c * CHUNK, CHUNK)],
                src_v.at[q], isems[q]).wait()
            pltpu.make_async_copy(
                dst_hbm.at[pl.ds(ebase + c * CHUNK, CHUNK)],
                dst_v.at[q], isems[q]).wait()

        def start_chunk(c, p, q):
            pltpu.async_copy(h_hbm.at[src_v.at[q]], rows_v.at[p], gsems[p])
            pltpu.async_copy(he_hbm.at[pl.ds(ebase + c * CHUNK, CHUNK)],
                             he_v.at[p], hsems[p])

        def wait_chunk(c, p, q):
            pltpu.make_async_copy(
                h_hbm.at[src_v.at[q]], rows_v.at[p], gsems[p]).wait()
            pltpu.make_async_copy(
                he_hbm.at[pl.ds(ebase + c * CHUNK, CHUNK)],
                he_v.at[p], hsems[p]).wait()

        def compute(c, p):
            rows = rows_v.at[p]
            hes = he_v.at[p]

            @plsc.parallel_loop(0, CHUNK, unroll=4)
            def _(r):
                rrow = rows.at[r]
                herow = hes.at[r]
                for v in range(D // LANES):
                    sl = pl.ds(v * LANES, LANES)
                    rrow[sl] = jnp.maximum(rrow[sl] + herow[sl], 0.0)

        def start_scatter(c, p, q):
            pltpu.async_copy(rows_v.at[p], agg_sh.at[dst_v.at[q]], ssems[p],
                             add=True)

        def wait_scatter(c, p, q):
            pltpu.make_async_copy(
                rows_v.at[p], agg_sh.at[dst_v.at[q]], ssems[p]).wait()

        start_idx(0, 0)
        wait_idx(0, 0)
        start_chunk(0, 0, 0)
        start_idx(1, 1)
        start_idx(2, 2)

        @pl.loop(0, nch, step=4)
        def _(j):
            for u in range(4):
                c = j + u
                p = u % 2
                q = u % 4
                pn = (u + 1) % 2
                qn = (u + 1) % 4
                qf = (u + 3) % 4

                @pl.when(c + 1 < nch)
                def _():
                    wait_idx(c + 1, qn)

                @pl.when(c >= 1)
                def _():
                    wait_scatter(c - 1, pn, qf)

                @pl.when(c + 1 < nch)
                def _():
                    start_chunk(c + 1, pn, qn)

                @pl.when(c + 3 < nch)
                def _():
                    start_idx(c + 3, qf)

                wait_chunk(c, p, q)
                compute(c, p)
                start_scatter(c, p, q)

        wait_scatter(nch - 1, 1, 3)
        plsc.subcore_barrier()
        pltpu.sync_copy(agg_sh.at[pl.ds(sid * RPS, RPS)],
                        out_hbm.at[cid, pl.ds(sid * RPS, RPS)])

    return k(h, src2, dst2, he)


def kernel(x, edge_index, edge_attr, batch, mol_desc, W_in, b_in,
           W1_0, b1_0, W2_0, b2_0, g_0, be_0,
           W1_1, b1_1, W2_1, b2_1, g_1, be_1,
           W1_2, b1_2, W2_2, b2_2, g_2, be_2,
           W_fc, b_fc):
    pad = EPAD - E
    src2 = jnp.pad(edge_index[0], (0, pad))
    dst2 = jnp.pad(edge_index[1], (0, pad), constant_values=N)
    ea = jnp.pad(edge_attr, ((0, pad), (0, 0)))
    batch2d = batch.reshape(1, N)

    layers = [(W1_0, b1_0, W2_0, b2_0, g_0, be_0),
              (W1_1, b1_1, W2_1, b2_1, g_1, be_1),
              (W1_2, b1_2, W2_2, b2_2, g_2, be_2)]

    h = _inproj(x, W_in, b_in)
    hes = [_edge_mlp(ea, W1, b1, W2, b2)
           for (W1, b1, W2, b2, _, _) in layers]
    for l in range(L):
        agg = _sc_messages(h, src2, dst2, hes[l])
        h = _bn_update(h, agg, layers[l][4], layers[l][5])

    wh = W_fc[:D, 0].reshape(1, D)
    wm = W_fc[D:, 0].reshape(1, MD)
    out = _pool_fc(h, batch2d, mol_desc, wh, wm, b_fc)
    return out[:, 0]


# speedup vs baseline: 1.0102x; 1.0102x over previous
import functools

import jax
import jax.numpy as jnp
from jax import lax
from jax.experimental import pallas as pl
from jax.experimental.pallas import tpu as pltpu
from jax.experimental.pallas import tpu_sc as plsc

N = 10000
E = 320000
D = 128
ED = 16
G = 64
MD = 17
L = 3

NC = 2
NS = 16
NW = NC * NS
CHUNK = 64
NCH_A = 224
NCH_B = 96
EPS_A = NCH_A * CHUNK
EPS_B = NCH_B * CHUNK
EPAD = NS * (EPS_A + EPS_B)
NPAD = 10240
RPS = NPAD // NS
EB = 2048
LANES = 16


def _inproj(x, W, b):
    def body(x_ref, w_ref, b_ref, o_ref):
        o_ref[...] = jnp.dot(x_ref[...], w_ref[...],
                             preferred_element_type=jnp.float32) + b_ref[...]
    return pl.pallas_call(
        body, out_shape=jax.ShapeDtypeStruct((N, D), jnp.float32),
    )(x, W, b.reshape(1, D))


def _edge_mlp(ea, W1, b1, W2, b2):
    def body(ea_ref, w1_ref, b1_ref, w2_ref, b2_ref, o_ref):
        t = jnp.maximum(
            jnp.dot(ea_ref[...], w1_ref[...],
                    preferred_element_type=jnp.float32) + b1_ref[...], 0.0)
        o_ref[...] = jnp.dot(t, w2_ref[...],
                             preferred_element_type=jnp.float32) + b2_ref[...]
    return pl.pallas_call(
        body,
        grid=(EPAD // EB,),
        in_specs=[
            pl.BlockSpec((EB, ED), lambda i: (i, 0)),
            pl.BlockSpec((ED, D), lambda i: (0, 0)),
            pl.BlockSpec((1, D), lambda i: (0, 0)),
            pl.BlockSpec((D, D), lambda i: (0, 0)),
            pl.BlockSpec((1, D), lambda i: (0, 0)),
        ],
        out_specs=pl.BlockSpec((EB, D), lambda i: (i, 0)),
        out_shape=jax.ShapeDtypeStruct((EPAD, D), jnp.float32),
    )(ea, W1, b1.reshape(1, D), W2, b2.reshape(1, D))


def _bn_update(h, agg, g, be):
    def body(h_ref, a_ref, g_ref, be_ref, o_ref):
        z = h_ref[...] + a_ref[0] + a_ref[1]
        mean = jnp.mean(z, axis=0, keepdims=True)
        zc = z - mean
        var = jnp.mean(zc * zc, axis=0, keepdims=True)
        y = g_ref[...] * zc * lax.rsqrt(var + 1e-5) + be_ref[...]
        o_ref[...] = jnp.maximum(y, 0.0)
    return pl.pallas_call(
        body,
        grid=(1,),
        in_specs=[
            pl.BlockSpec((N, D), lambda i: (0, 0)),
            pl.BlockSpec((NC, N, D), lambda i: (0, 0, 0)),
            pl.BlockSpec((1, D), lambda i: (0, 0)),
            pl.BlockSpec((1, D), lambda i: (0, 0)),
        ],
        out_specs=pl.BlockSpec((N, D), lambda i: (0, 0)),
        out_shape=jax.ShapeDtypeStruct((N, D), jnp.float32),
    )(h, agg, g.reshape(1, D), be.reshape(1, D))


def _pool_fc(h, batch2d, mol_desc, wh, wm, bf):
    def body(h_ref, b_ref, md_ref, wh_ref, wm_ref, bf_ref, o_ref):
        gids = lax.broadcasted_iota(jnp.int32, (G, N), 0)
        M = (b_ref[...] == gids).astype(jnp.float32)
        sums = jnp.dot(M, h_ref[...], preferred_element_type=jnp.float32)
        counts = jnp.sum(M, axis=1, keepdims=True)
        pooled = sums / jnp.maximum(counts, 1.0)
        o_ref[...] = (jnp.sum(pooled * wh_ref[...], axis=1, keepdims=True)
                      + jnp.sum(md_ref[...] * wm_ref[...], axis=1, keepdims=True)
                      + bf_ref[0, 0])
    return pl.pallas_call(
        body, out_shape=jax.ShapeDtypeStruct((G, 1), jnp.float32),
    )(h, batch2d, mol_desc, wh, wm, bf.reshape(1, 1))


def _sc_messages(h, src2, dst2, he):
    mesh = plsc.VectorSubcoreMesh(core_axis_name="c", subcore_axis_name="s",
                                  num_cores=2)

    @functools.partial(
        pl.kernel,
        out_type=jax.ShapeDtypeStruct((NC, NPAD, D), jnp.float32),
        mesh=mesh,
        scratch_types=[
            pltpu.VMEM((4, CHUNK), jnp.int32),
            pltpu.VMEM((4, CHUNK), jnp.int32),
            pltpu.VMEM((2, CHUNK, D), jnp.float32),
            pltpu.VMEM((2, CHUNK, D), jnp.float32),
            pltpu.VMEM_SHARED((NPAD, D), jnp.float32),
            pltpu.SemaphoreType.DMA,
            pltpu.SemaphoreType.DMA,
            pltpu.SemaphoreType.DMA,
            pltpu.SemaphoreType.DMA,
            pltpu.SemaphoreType.DMA,
            pltpu.SemaphoreType.DMA,
            pltpu.SemaphoreType.DMA,
            pltpu.SemaphoreType.DMA,
            pltpu.SemaphoreType.DMA,
            pltpu.SemaphoreType.DMA,
        ],
    )
    def k(h_hbm, src_hbm, dst_hbm, he_hbm, out_hbm,
          src_v, dst_v, rows_v, he_v, agg_sh,
          is0, is1, is2, is3, gs0, gs1, hs0, hs1, ss0, ss1):
        cid = lax.axis_index("c")
        sid = lax.axis_index("s")
        on_a = cid == 0
        nch = jnp.where(on_a, NCH_A, NCH_B)
        ebase = jnp.where(on_a, sid * EPS_A, NS * EPS_A + sid * EPS_B)
        isems = (is0, is1, is2, is3)
        gsems = (gs0, gs1)
        hsems = (hs0, hs1)
        ssems = (ss0, ss1)

        zbuf = rows_v.at[0]

        @pl.loop(0, CHUNK)
        def _(r):
            zrow = zbuf.at[r]
            for v in range(D // LANES):
                zrow[pl.ds(v * LANES, LANES)] = jnp.zeros((LANES,),
                                                          jnp.float32)

        @pl.loop(0, RPS // CHUNK)
        def _(t):
            pltpu.sync_copy(
                zbuf, agg_sh.at[pl.ds(sid * RPS + t * CHUNK, CHUNK)])
        plsc.subcore_barrier()

        def start_idx(c, q):
            pltpu.async_copy(src_hbm.at[pl.ds(ebase + c * CHUNK, CHUNK)],
                             src_v.at[q], isems[q])
            pltpu.async_copy(dst_hbm.at[pl.ds(ebase + c * CHUNK, CHUNK)],
                             dst_v.at[q], isems[q])

        def wait_idx(c, q):
            pltpu.make_async_copy(
                src_hbm.at[pl.ds(ebase + c * CHUNK, CHUNK)],
                src_v.at[q], isems[q]).wait()
            pltpu.make_async_copy(
                dst_hbm.at[pl.ds(ebase + c * CHUNK, CHUNK)],
                dst_v.at[q], isems[q]).wait()

        def start_chunk(c, p, q):
            pltpu.async_copy(h_hbm.at[src_v.at[q]], rows_v.at[p], gsems[p])
            pltpu.async_copy(he_hbm.at[pl.ds(ebase + c * CHUNK, CHUNK)],
                             he_v.at[p], hsems[p])

        def wait_chunk(c, p, q):
            pltpu.make_async_copy(
                h_hbm.at[src_v.at[q]], rows_v.at[p], gsems[p]).wait()
            pltpu.make_async_copy(
                he_hbm.at[pl.ds(ebase + c * CHUNK, CHUNK)],
                he_v.at[p], hsems[p]).wait()

        def compute(c, p):
            rows = rows_v.at[p]
            hes = he_v.at[p]

            @plsc.parallel_loop(0, CHUNK, unroll=4)
            def _(r):
                rrow = rows.at[r]
                herow = hes.at[r]
                for v in range(D // LANES):
                    sl = pl.ds(v * LANES, LANES)
                    rrow[sl] = jnp.maximum(rrow[sl] + herow[sl], 0.0)

        def start_scatter(c, p, q):
            pltpu.async_copy(rows_v.at[p], agg_sh.at[dst_v.at[q]], ssems[p],
                             add=True)

        def wait_scatter(c, p, q):
            pltpu.make_async_copy(
                rows_v.at[p], agg_sh.at[dst_v.at[q]], ssems[p]).wait()

        start_idx(0, 0)
        wait_idx(0, 0)
        start_chunk(0, 0, 0)
        start_idx(1, 1)
        start_idx(2, 2)

        @pl.loop(0, nch, step=4)
        def _(j):
            for u in range(4):
                c = j + u
                p = u % 2
                q = u % 4
                pn = (u + 1) % 2
                qn = (u + 1) % 4
                qf = (u + 3) % 4

                @pl.when(c + 1 < nch)
                def _():
                    wait_idx(c + 1, qn)

                @pl.when(c >= 1)
                def _():
                    wait_scatter(c - 1, pn, qf)

                @pl.when(c + 1 < nch)
                def _():
                    start_chunk(c + 1, pn, qn)

                @pl.when(c + 3 < nch)
                def _():
                    start_idx(c + 3, qf)

                wait_chunk(c, p, q)
                compute(c, p)
                start_scatter(c, p, q)

        wait_scatter(nch - 1, 1, 3)
        plsc.subcore_barrier()
        pltpu.sync_copy(agg_sh.at[pl.ds(sid * RPS, RPS)],
                        out_hbm.at[cid, pl.ds(sid * RPS, RPS)])

    return k(h, src2, dst2, he)


def kernel(x, edge_index, edge_attr, batch, mol_desc, W_in, b_in,
           W1_0, b1_0, W2_0, b2_0, g_0, be_0,
           W1_1, b1_1, W2_1, b2_1, g_1, be_1,
           W1_2, b1_2, W2_2, b2_2, g_2, be_2,
           W_fc, b_fc):
    pad = EPAD - E
    src2 = jnp.pad(edge_index[0], (0, pad))
    dst2 = jnp.pad(edge_index[1], (0, pad), constant_values=N)
    ea = jnp.pad(edge_attr, ((0, pad), (0, 0)))
    batch2d = batch.reshape(1, N)

    layers = [(W1_0, b1_0, W2_0, b2_0, g_0, be_0),
              (W1_1, b1_1, W2_1, b2_1, g_1, be_1),
              (W1_2, b1_2, W2_2, b2_2, g_2, be_2)]

    h = _inproj(x, W_in, b_in)
    hes = [_edge_mlp(ea, W1, b1, W2, b2)
           for (W1, b1, W2, b2, _, _) in layers]
    for l in range(L):
        agg = _sc_messages(h, src2, dst2, hes[l])
        h = _bn_update(h, agg, layers[l][4], layers[l][5])

    wh = W_fc[:D, 0].reshape(1, D)
    wm = W_fc[D:, 0].reshape(1, MD)
    out = _pool_fc(h, batch2d, mol_desc, wh, wm, b_fc)
    return out[:, 0]


# speedup vs baseline: 1.0170x; 1.0068x over previous
import functools

import jax
import jax.numpy as jnp
from jax import lax
from jax.experimental import pallas as pl
from jax.experimental.pallas import tpu as pltpu
from jax.experimental.pallas import tpu_sc as plsc

N = 10000
E = 320000
D = 128
ED = 16
G = 64
MD = 17
L = 3

NC = 2
NS = 16
NW = NC * NS
CHUNK = 64
NCH_A = 240
NCH_B = 80
EPS_A = NCH_A * CHUNK
EPS_B = NCH_B * CHUNK
EPAD = NS * (EPS_A + EPS_B)
NPAD = 10240
RPS = NPAD // NS
EB = 2048
LANES = 16


def _inproj(x, W, b):
    def body(x_ref, w_ref, b_ref, o_ref):
        o_ref[...] = jnp.dot(x_ref[...], w_ref[...],
                             preferred_element_type=jnp.float32) + b_ref[...]
    return pl.pallas_call(
        body, out_shape=jax.ShapeDtypeStruct((N, D), jnp.float32),
    )(x, W, b.reshape(1, D))


def _edge_mlp(ea, W1, b1, W2, b2):
    def body(ea_ref, w1_ref, b1_ref, w2_ref, b2_ref, o_ref):
        t = jnp.maximum(
            jnp.dot(ea_ref[...], w1_ref[...],
                    preferred_element_type=jnp.float32) + b1_ref[...], 0.0)
        o_ref[...] = jnp.dot(t, w2_ref[...],
                             preferred_element_type=jnp.float32) + b2_ref[...]
    return pl.pallas_call(
        body,
        grid=(EPAD // EB,),
        in_specs=[
            pl.BlockSpec((EB, ED), lambda i: (i, 0)),
            pl.BlockSpec((ED, D), lambda i: (0, 0)),
            pl.BlockSpec((1, D), lambda i: (0, 0)),
            pl.BlockSpec((D, D), lambda i: (0, 0)),
            pl.BlockSpec((1, D), lambda i: (0, 0)),
        ],
        out_specs=pl.BlockSpec((EB, D), lambda i: (i, 0)),
        out_shape=jax.ShapeDtypeStruct((EPAD, D), jnp.float32),
    )(ea, W1, b1.reshape(1, D), W2, b2.reshape(1, D))


def _bn_update(h, agg, g, be):
    def body(h_ref, a_ref, g_ref, be_ref, o_ref):
        z = h_ref[...] + a_ref[0] + a_ref[1]
        mean = jnp.mean(z, axis=0, keepdims=True)
        zc = z - mean
        var = jnp.mean(zc * zc, axis=0, keepdims=True)
        y = g_ref[...] * zc * lax.rsqrt(var + 1e-5) + be_ref[...]
        o_ref[...] = jnp.maximum(y, 0.0)
    return pl.pallas_call(
        body,
        grid=(1,),
        in_specs=[
            pl.BlockSpec((N, D), lambda i: (0, 0)),
            pl.BlockSpec((NC, N, D), lambda i: (0, 0, 0)),
            pl.BlockSpec((1, D), lambda i: (0, 0)),
            pl.BlockSpec((1, D), lambda i: (0, 0)),
        ],
        out_specs=pl.BlockSpec((N, D), lambda i: (0, 0)),
        out_shape=jax.ShapeDtypeStruct((N, D), jnp.float32),
    )(h, agg, g.reshape(1, D), be.reshape(1, D))


def _pool_fc(h, batch2d, mol_desc, wh, wm, bf):
    def body(h_ref, b_ref, md_ref, wh_ref, wm_ref, bf_ref, o_ref):
        gids = lax.broadcasted_iota(jnp.int32, (G, N), 0)
        M = (b_ref[...] == gids).astype(jnp.float32)
        sums = jnp.dot(M, h_ref[...], preferred_element_type=jnp.float32)
        counts = jnp.sum(M, axis=1, keepdims=True)
        pooled = sums / jnp.maximum(counts, 1.0)
        o_ref[...] = (jnp.sum(pooled * wh_ref[...], axis=1, keepdims=True)
                      + jnp.sum(md_ref[...] * wm_ref[...], axis=1, keepdims=True)
                      + bf_ref[0, 0])
    return pl.pallas_call(
        body, out_shape=jax.ShapeDtypeStruct((G, 1), jnp.float32),
    )(h, batch2d, mol_desc, wh, wm, bf.reshape(1, 1))


def _sc_messages(h, src2, dst2, he):
    mesh = plsc.VectorSubcoreMesh(core_axis_name="c", subcore_axis_name="s",
                                  num_cores=2)

    @functools.partial(
        pl.kernel,
        out_type=jax.ShapeDtypeStruct((NC, NPAD, D), jnp.float32),
        mesh=mesh,
        scratch_types=[
            pltpu.VMEM((4, CHUNK), jnp.int32),
            pltpu.VMEM((4, CHUNK), jnp.int32),
            pltpu.VMEM((2, CHUNK, D), jnp.float32),
            pltpu.VMEM((2, CHUNK, D), jnp.float32),
            pltpu.VMEM_SHARED((NPAD, D), jnp.float32),
            pltpu.SemaphoreType.DMA,
            pltpu.SemaphoreType.DMA,
            pltpu.SemaphoreType.DMA,
            pltpu.SemaphoreType.DMA,
            pltpu.SemaphoreType.DMA,
            pltpu.SemaphoreType.DMA,
            pltpu.SemaphoreType.DMA,
            pltpu.SemaphoreType.DMA,
            pltpu.SemaphoreType.DMA,
            pltpu.SemaphoreType.DMA,
        ],
    )
    def k(h_hbm, src_hbm, dst_hbm, he_hbm, out_hbm,
          src_v, dst_v, rows_v, he_v, agg_sh,
          is0, is1, is2, is3, gs0, gs1, hs0, hs1, ss0, ss1):
        cid = lax.axis_index("c")
        sid = lax.axis_index("s")
        on_a = cid == 0
        nch = jnp.where(on_a, NCH_A, NCH_B)
        ebase = jnp.where(on_a, sid * EPS_A, NS * EPS_A + sid * EPS_B)
        isems = (is0, is1, is2, is3)
        gsems = (gs0, gs1)
        hsems = (hs0, hs1)
        ssems = (ss0, ss1)

        zbuf = rows_v.at[0]

        @pl.loop(0, CHUNK)
        def _(r):
            zrow = zbuf.at[r]
            for v in range(D // LANES):
                zrow[pl.ds(v * LANES, LANES)] = jnp.zeros((LANES,),
                                                          jnp.float32)

        @pl.loop(0, RPS // CHUNK)
        def _(t):
            pltpu.sync_copy(
                zbuf, agg_sh.at[pl.ds(sid * RPS + t * CHUNK, CHUNK)])
        plsc.subcore_barrier()

        def start_idx(c, q):
            pltpu.async_copy(src_hbm.at[pl.ds(ebase + c * CHUNK, CHUNK)],
                             src_v.at[q], isems[q])
            pltpu.async_copy(dst_hbm.at[pl.ds(ebase + c * CHUNK, CHUNK)],
                             dst_v.at[q], isems[q])

        def wait_idx(c, q):
            pltpu.make_async_copy(
                src_hbm.at[pl.ds(ebase + c * CHUNK, CHUNK)],
                src_v.at[q], isems[q]).wait()
            pltpu.make_async_copy(
                dst_hbm.at[pl.ds(ebase + c * CHUNK, CHUNK)],
                dst_v.at[q], isems[q]).wait()

        def start_chunk(c, p, q):
            pltpu.async_copy(h_hbm.at[src_v.at[q]], rows_v.at[p], gsems[p])
            pltpu.async_copy(he_hbm.at[pl.ds(ebase + c * CHUNK, CHUNK)],
                             he_v.at[p], hsems[p])

        def wait_chunk(c, p, q):
            pltpu.make_async_copy(
                h_hbm.at[src_v.at[q]], rows_v.at[p], gsems[p]).wait()
            pltpu.make_async_copy(
                he_hbm.at[pl.ds(ebase + c * CHUNK, CHUNK)],
                he_v.at[p], hsems[p]).wait()

        def compute(c, p):
            rows = rows_v.at[p]
            hes = he_v.at[p]

            @plsc.parallel_loop(0, CHUNK, unroll=4)
            def _(r):
                rrow = rows.at[r]
                herow = hes.at[r]
                for v in range(D // LANES):
                    sl = pl.ds(v * LANES, LANES)
                    rrow[sl] = jnp.maximum(rrow[sl] + herow[sl], 0.0)

        def start_scatter(c, p, q):
            pltpu.async_copy(rows_v.at[p], agg_sh.at[dst_v.at[q]], ssems[p],
                             add=True)

        def wait_scatter(c, p, q):
            pltpu.make_async_copy(
                rows_v.at[p], agg_sh.at[dst_v.at[q]], ssems[p]).wait()

        start_idx(0, 0)
        wait_idx(0, 0)
        start_chunk(0, 0, 0)
        start_idx(1, 1)
        start_idx(2, 2)

        @pl.loop(0, nch, step=4)
        def _(j):
            for u in range(4):
                c = j + u
                p = u % 2
                q = u % 4
                pn = (u + 1) % 2
                qn = (u + 1) % 4
                qf = (u + 3) % 4

                @pl.when(c + 1 < nch)
                def _():
                    wait_idx(c + 1, qn)

                @pl.when(c >= 1)
                def _():
                    wait_scatter(c - 1, pn, qf)

                @pl.when(c + 1 < nch)
                def _():
                    start_chunk(c + 1, pn, qn)

                @pl.when(c + 3 < nch)
                def _():
                    start_idx(c + 3, qf)

                wait_chunk(c, p, q)
                compute(c, p)
                start_scatter(c, p, q)

        wait_scatter(nch - 1, 1, 3)
        plsc.subcore_barrier()
        pltpu.sync_copy(agg_sh.at[pl.ds(sid * RPS, RPS)],
                        out_hbm.at[cid, pl.ds(sid * RPS, RPS)])

    return k(h, src2, dst2, he)


def kernel(x, edge_index, edge_attr, batch, mol_desc, W_in, b_in,
           W1_0, b1_0, W2_0, b2_0, g_0, be_0,
           W1_1, b1_1, W2_1, b2_1, g_1, be_1,
           W1_2, b1_2, W2_2, b2_2, g_2, be_2,
           W_fc, b_fc):
    pad = EPAD - E
    src2 = jnp.pad(edge_index[0], (0, pad))
    dst2 = jnp.pad(edge_index[1], (0, pad), constant_values=N)
    ea = jnp.pad(edge_attr, ((0, pad), (0, 0)))
    batch2d = batch.reshape(1, N)

    layers = [(W1_0, b1_0, W2_0, b2_0, g_0, be_0),
              (W1_1, b1_1, W2_1, b2_1, g_1, be_1),
              (W1_2, b1_2, W2_2, b2_2, g_2, be_2)]

    h = _inproj(x, W_in, b_in)
    hes = [_edge_mlp(ea, W1, b1, W2, b2)
           for (W1, b1, W2, b2, _, _) in layers]
    for l in range(L):
        agg = _sc_messages(h, src2, dst2, hes[l])
        h = _bn_update(h, agg, layers[l][4], layers[l][5])

    wh = W_fc[:D, 0].reshape(1, D)
    wm = W_fc[D:, 0].reshape(1, MD)
    out = _pool_fc(h, batch2d, mol_desc, wh, wm, b_fc)
    return out[:, 0]


# speedup vs baseline: 1.0217x; 1.0046x over previous
import functools

import jax
import jax.numpy as jnp
from jax import lax
from jax.experimental import pallas as pl
from jax.experimental.pallas import tpu as pltpu
from jax.experimental.pallas import tpu_sc as plsc

N = 10000
E = 320000
D = 128
ED = 16
G = 64
MD = 17
L = 3

NC = 2
NS = 16
NW = NC * NS
CHUNK = 64
NCH_A = 256
NCH_B = 64
EPS_A = NCH_A * CHUNK
EPS_B = NCH_B * CHUNK
EPAD = NS * (EPS_A + EPS_B)
NPAD = 10240
RPS = NPAD // NS
EB = 2048
LANES = 16


def _inproj(x, W, b):
    def body(x_ref, w_ref, b_ref, o_ref):
        o_ref[...] = jnp.dot(x_ref[...], w_ref[...],
                             preferred_element_type=jnp.float32) + b_ref[...]
    return pl.pallas_call(
        body, out_shape=jax.ShapeDtypeStruct((N, D), jnp.float32),
    )(x, W, b.reshape(1, D))


def _edge_mlp(ea, W1, b1, W2, b2):
    def body(ea_ref, w1_ref, b1_ref, w2_ref, b2_ref, o_ref):
        t = jnp.maximum(
            jnp.dot(ea_ref[...], w1_ref[...],
                    preferred_element_type=jnp.float32) + b1_ref[...], 0.0)
        o_ref[...] = jnp.dot(t, w2_ref[...],
                             preferred_element_type=jnp.float32) + b2_ref[...]
    return pl.pallas_call(
        body,
        grid=(EPAD // EB,),
        in_specs=[
            pl.BlockSpec((EB, ED), lambda i: (i, 0)),
            pl.BlockSpec((ED, D), lambda i: (0, 0)),
            pl.BlockSpec((1, D), lambda i: (0, 0)),
            pl.BlockSpec((D, D), lambda i: (0, 0)),
            pl.BlockSpec((1, D), lambda i: (0, 0)),
        ],
        out_specs=pl.BlockSpec((EB, D), lambda i: (i, 0)),
        out_shape=jax.ShapeDtypeStruct((EPAD, D), jnp.float32),
    )(ea, W1, b1.reshape(1, D), W2, b2.reshape(1, D))


def _bn_update(h, agg, g, be):
    def body(h_ref, a_ref, g_ref, be_ref, o_ref):
        z = h_ref[...] + a_ref[0] + a_ref[1]
        mean = jnp.mean(z, axis=0, keepdims=True)
        zc = z - mean
        var = jnp.mean(zc * zc, axis=0, keepdims=True)
        y = g_ref[...] * zc * lax.rsqrt(var + 1e-5) + be_ref[...]
        o_ref[...] = jnp.maximum(y, 0.0)
    return pl.pallas_call(
        body,
        grid=(1,),
        in_specs=[
            pl.BlockSpec((N, D), lambda i: (0, 0)),
            pl.BlockSpec((NC, N, D), lambda i: (0, 0, 0)),
            pl.BlockSpec((1, D), lambda i: (0, 0)),
            pl.BlockSpec((1, D), lambda i: (0, 0)),
        ],
        out_specs=pl.BlockSpec((N, D), lambda i: (0, 0)),
        out_shape=jax.ShapeDtypeStruct((N, D), jnp.float32),
    )(h, agg, g.reshape(1, D), be.reshape(1, D))


def _pool_fc(h, batch2d, mol_desc, wh, wm, bf):
    def body(h_ref, b_ref, md_ref, wh_ref, wm_ref, bf_ref, o_ref):
        gids = lax.broadcasted_iota(jnp.int32, (G, N), 0)
        M = (b_ref[...] == gids).astype(jnp.float32)
        sums = jnp.dot(M, h_ref[...], preferred_element_type=jnp.float32)
        counts = jnp.sum(M, axis=1, keepdims=True)
        pooled = sums / jnp.maximum(counts, 1.0)
        o_ref[...] = (jnp.sum(pooled * wh_ref[...], axis=1, keepdims=True)
                      + jnp.sum(md_ref[...] * wm_ref[...], axis=1, keepdims=True)
                      + bf_ref[0, 0])
    return pl.pallas_call(
        body, out_shape=jax.ShapeDtypeStruct((G, 1), jnp.float32),
    )(h, batch2d, mol_desc, wh, wm, bf.reshape(1, 1))


def _sc_messages(h, src2, dst2, he):
    mesh = plsc.VectorSubcoreMesh(core_axis_name="c", subcore_axis_name="s",
                                  num_cores=2)

    @functools.partial(
        pl.kernel,
        out_type=jax.ShapeDtypeStruct((NC, NPAD, D), jnp.float32),
        mesh=mesh,
        scratch_types=[
            pltpu.VMEM((4, CHUNK), jnp.int32),
            pltpu.VMEM((4, CHUNK), jnp.int32),
            pltpu.VMEM((2, CHUNK, D), jnp.float32),
            pltpu.VMEM((2, CHUNK, D), jnp.float32),
            pltpu.VMEM_SHARED((NPAD, D), jnp.float32),
            pltpu.SemaphoreType.DMA,
            pltpu.SemaphoreType.DMA,
            pltpu.SemaphoreType.DMA,
            pltpu.SemaphoreType.DMA,
            pltpu.SemaphoreType.DMA,
            pltpu.SemaphoreType.DMA,
            pltpu.SemaphoreType.DMA,
            pltpu.SemaphoreType.DMA,
            pltpu.SemaphoreType.DMA,
            pltpu.SemaphoreType.DMA,
        ],
    )
    def k(h_hbm, src_hbm, dst_hbm, he_hbm, out_hbm,
          src_v, dst_v, rows_v, he_v, agg_sh,
          is0, is1, is2, is3, gs0, gs1, hs0, hs1, ss0, ss1):
        cid = lax.axis_index("c")
        sid = lax.axis_index("s")
        on_a = cid == 0
        nch = jnp.where(on_a, NCH_A, NCH_B)
        ebase = jnp.where(on_a, sid * EPS_A, NS * EPS_A + sid * EPS_B)
        isems = (is0, is1, is2, is3)
        gsems = (gs0, gs1)
        hsems = (hs0, hs1)
        ssems = (ss0, ss1)

        zbuf = rows_v.at[0]

        @pl.loop(0, CHUNK)
        def _(r):
            zrow = zbuf.at[r]
            for v in range(D // LANES):
                zrow[pl.ds(v * LANES, LANES)] = jnp.zeros((LANES,),
                                                          jnp.float32)

        @pl.loop(0, RPS // CHUNK)
        def _(t):
            pltpu.sync_copy(
                zbuf, agg_sh.at[pl.ds(sid * RPS + t * CHUNK, CHUNK)])
        plsc.subcore_barrier()

        def start_idx(c, q):
            pltpu.async_copy(src_hbm.at[pl.ds(ebase + c * CHUNK, CHUNK)],
                             src_v.at[q], isems[q])
            pltpu.async_copy(dst_hbm.at[pl.ds(ebase + c * CHUNK, CHUNK)],
                             dst_v.at[q], isems[q])

        def wait_idx(c, q):
            pltpu.make_async_copy(
                src_hbm.at[pl.ds(ebase + c * CHUNK, CHUNK)],
                src_v.at[q], isems[q]).wait()
            pltpu.make_async_copy(
                dst_hbm.at[pl.ds(ebase + c * CHUNK, CHUNK)],
                dst_v.at[q], isems[q]).wait()

        def start_chunk(c, p, q):
            pltpu.async_copy(h_hbm.at[src_v.at[q]], rows_v.at[p], gsems[p])
            pltpu.async_copy(he_hbm.at[pl.ds(ebase + c * CHUNK, CHUNK)],
                             he_v.at[p], hsems[p])

        def wait_chunk(c, p, q):
            pltpu.make_async_copy(
                h_hbm.at[src_v.at[q]], rows_v.at[p], gsems[p]).wait()
            pltpu.make_async_copy(
                he_hbm.at[pl.ds(ebase + c * CHUNK, CHUNK)],
                he_v.at[p], hsems[p]).wait()

        def compute(c, p):
            rows = rows_v.at[p]
            hes = he_v.at[p]

            @plsc.parallel_loop(0, CHUNK, unroll=4)
            def _(r):
                rrow = rows.at[r]
                herow = hes.at[r]
                for v in range(D // LANES):
                    sl = pl.ds(v * LANES, LANES)
                    rrow[sl] = jnp.maximum(rrow[sl] + herow[sl], 0.0)

        def start_scatter(c, p, q):
            pltpu.async_copy(rows_v.at[p], agg_sh.at[dst_v.at[q]], ssems[p],
                             add=True)

        def wait_scatter(c, p, q):
            pltpu.make_async_copy(
                rows_v.at[p], agg_sh.at[dst_v.at[q]], ssems[p]).wait()

        start_idx(0, 0)
        wait_idx(0, 0)
        start_chunk(0, 0, 0)
        start_idx(1, 1)
        start_idx(2, 2)

        @pl.loop(0, nch, step=4)
        def _(j):
            for u in range(4):
                c = j + u
                p = u % 2
                q = u % 4
                pn = (u + 1) % 2
                qn = (u + 1) % 4
                qf = (u + 3) % 4

                @pl.when(c + 1 < nch)
                def _():
                    wait_idx(c + 1, qn)

                @pl.when(c >= 1)
                def _():
                    wait_scatter(c - 1, pn, qf)

                @pl.when(c + 1 < nch)
                def _():
                    start_chunk(c + 1, pn, qn)

                @pl.when(c + 3 < nch)
                def _():
                    start_idx(c + 3, qf)

                wait_chunk(c, p, q)
                compute(c, p)
                start_scatter(c, p, q)

        wait_scatter(nch - 1, 1, 3)
        plsc.subcore_barrier()
        pltpu.sync_copy(agg_sh.at[pl.ds(sid * RPS, RPS)],
                        out_hbm.at[cid, pl.ds(sid * RPS, RPS)])

    return k(h, src2, dst2, he)


def kernel(x, edge_index, edge_attr, batch, mol_desc, W_in, b_in,
           W1_0, b1_0, W2_0, b2_0, g_0, be_0,
           W1_1, b1_1, W2_1, b2_1, g_1, be_1,
           W1_2, b1_2, W2_2, b2_2, g_2, be_2,
           W_fc, b_fc):
    pad = EPAD - E
    src2 = jnp.pad(edge_index[0], (0, pad))
    dst2 = jnp.pad(edge_index[1], (0, pad), constant_values=N)
    ea = jnp.pad(edge_attr, ((0, pad), (0, 0)))
    batch2d = batch.reshape(1, N)

    layers = [(W1_0, b1_0, W2_0, b2_0, g_0, be_0),
              (W1_1, b1_1, W2_1, b2_1, g_1, be_1),
              (W1_2, b1_2, W2_2, b2_2, g_2, be_2)]

    h = _inproj(x, W_in, b_in)
    hes = [_edge_mlp(ea, W1, b1, W2, b2)
           for (W1, b1, W2, b2, _, _) in layers]
    for l in range(L):
        agg = _sc_messages(h, src2, dst2, hes[l])
        h = _bn_update(h, agg, layers[l][4], layers[l][5])

    wh = W_fc[:D, 0].reshape(1, D)
    wm = W_fc[D:, 0].reshape(1, MD)
    out = _pool_fc(h, batch2d, mol_desc, wh, wm, b_fc)
    return out[:, 0]


# speedup vs baseline: 1.0392x; 1.0171x over previous
import functools

import jax
import jax.numpy as jnp
from jax import lax
from jax.experimental import pallas as pl
from jax.experimental.pallas import tpu as pltpu
from jax.experimental.pallas import tpu_sc as plsc

N = 10000
E = 320000
D = 128
ED = 16
G = 64
MD = 17
L = 3

NC = 2
NS = 16
NW = NC * NS
CHUNK = 64
NCH_A = 272
NCH_B = 48
EPS_A = NCH_A * CHUNK
EPS_B = NCH_B * CHUNK
EPAD = NS * (EPS_A + EPS_B)
NPAD = 10240
RPS = NPAD // NS
EB = 2048
LANES = 16


def _inproj(x, W, b):
    def body(x_ref, w_ref, b_ref, o_ref):
        o_ref[...] = jnp.dot(x_ref[...], w_ref[...],
                             preferred_element_type=jnp.float32) + b_ref[...]
    return pl.pallas_call(
        body, out_shape=jax.ShapeDtypeStruct((N, D), jnp.float32),
    )(x, W, b.reshape(1, D))


def _edge_mlp(ea, W1, b1, W2, b2):
    def body(ea_ref, w1_ref, b1_ref, w2_ref, b2_ref, o_ref):
        t = jnp.maximum(
            jnp.dot(ea_ref[...], w1_ref[...],
                    preferred_element_type=jnp.float32) + b1_ref[...], 0.0)
        o_ref[...] = jnp.dot(t, w2_ref[...],
                             preferred_element_type=jnp.float32) + b2_ref[...]
    return pl.pallas_call(
        body,
        grid=(EPAD // EB,),
        in_specs=[
            pl.BlockSpec((EB, ED), lambda i: (i, 0)),
            pl.BlockSpec((ED, D), lambda i: (0, 0)),
            pl.BlockSpec((1, D), lambda i: (0, 0)),
            pl.BlockSpec((D, D), lambda i: (0, 0)),
            pl.BlockSpec((1, D), lambda i: (0, 0)),
        ],
        out_specs=pl.BlockSpec((EB, D), lambda i: (i, 0)),
        out_shape=jax.ShapeDtypeStruct((EPAD, D), jnp.float32),
    )(ea, W1, b1.reshape(1, D), W2, b2.reshape(1, D))


def _bn_update(h, agg, g, be):
    def body(h_ref, a_ref, g_ref, be_ref, o_ref):
        z = h_ref[...] + a_ref[0] + a_ref[1]
        mean = jnp.mean(z, axis=0, keepdims=True)
        zc = z - mean
        var = jnp.mean(zc * zc, axis=0, keepdims=True)
        y = g_ref[...] * zc * lax.rsqrt(var + 1e-5) + be_ref[...]
        o_ref[...] = jnp.maximum(y, 0.0)
    return pl.pallas_call(
        body,
        grid=(1,),
        in_specs=[
            pl.BlockSpec((N, D), lambda i: (0, 0)),
            pl.BlockSpec((NC, N, D), lambda i: (0, 0, 0)),
            pl.BlockSpec((1, D), lambda i: (0, 0)),
            pl.BlockSpec((1, D), lambda i: (0, 0)),
        ],
        out_specs=pl.BlockSpec((N, D), lambda i: (0, 0)),
        out_shape=jax.ShapeDtypeStruct((N, D), jnp.float32),
    )(h, agg, g.reshape(1, D), be.reshape(1, D))


def _pool_fc(h, batch2d, mol_desc, wh, wm, bf):
    def body(h_ref, b_ref, md_ref, wh_ref, wm_ref, bf_ref, o_ref):
        gids = lax.broadcasted_iota(jnp.int32, (G, N), 0)
        M = (b_ref[...] == gids).astype(jnp.float32)
        sums = jnp.dot(M, h_ref[...], preferred_element_type=jnp.float32)
        counts = jnp.sum(M, axis=1, keepdims=True)
        pooled = sums / jnp.maximum(counts, 1.0)
        o_ref[...] = (jnp.sum(pooled * wh_ref[...], axis=1, keepdims=True)
                      + jnp.sum(md_ref[...] * wm_ref[...], axis=1, keepdims=True)
                      + bf_ref[0, 0])
    return pl.pallas_call(
        body, out_shape=jax.ShapeDtypeStruct((G, 1), jnp.float32),
    )(h, batch2d, mol_desc, wh, wm, bf.reshape(1, 1))


def _sc_messages(h, src2, dst2, he):
    mesh = plsc.VectorSubcoreMesh(core_axis_name="c", subcore_axis_name="s",
                                  num_cores=2)

    @functools.partial(
        pl.kernel,
        out_type=jax.ShapeDtypeStruct((NC, NPAD, D), jnp.float32),
        mesh=mesh,
        scratch_types=[
            pltpu.VMEM((4, CHUNK), jnp.int32),
            pltpu.VMEM((4, CHUNK), jnp.int32),
            pltpu.VMEM((2, CHUNK, D), jnp.float32),
            pltpu.VMEM((2, CHUNK, D), jnp.float32),
            pltpu.VMEM_SHARED((NPAD, D), jnp.float32),
            pltpu.SemaphoreType.DMA,
            pltpu.SemaphoreType.DMA,
            pltpu.SemaphoreType.DMA,
            pltpu.SemaphoreType.DMA,
            pltpu.SemaphoreType.DMA,
            pltpu.SemaphoreType.DMA,
            pltpu.SemaphoreType.DMA,
            pltpu.SemaphoreType.DMA,
            pltpu.SemaphoreType.DMA,
            pltpu.SemaphoreType.DMA,
        ],
    )
    def k(h_hbm, src_hbm, dst_hbm, he_hbm, out_hbm,
          src_v, dst_v, rows_v, he_v, agg_sh,
          is0, is1, is2, is3, gs0, gs1, hs0, hs1, ss0, ss1):
        cid = lax.axis_index("c")
        sid = lax.axis_index("s")
        on_a = cid == 0
        nch = jnp.where(on_a, NCH_A, NCH_B)
        ebase = jnp.where(on_a, sid * EPS_A, NS * EPS_A + sid * EPS_B)
        isems = (is0, is1, is2, is3)
        gsems = (gs0, gs1)
        hsems = (hs0, hs1)
        ssems = (ss0, ss1)

        zbuf = rows_v.at[0]

        @pl.loop(0, CHUNK)
        def _(r):
            zrow = zbuf.at[r]
            for v in range(D // LANES):
                zrow[pl.ds(v * LANES, LANES)] = jnp.zeros((LANES,),
                                                          jnp.float32)

        @pl.loop(0, RPS // CHUNK)
        def _(t):
            pltpu.sync_copy(
                zbuf, agg_sh.at[pl.ds(sid * RPS + t * CHUNK, CHUNK)])
        plsc.subcore_barrier()

        def start_idx(c, q):
            pltpu.async_copy(src_hbm.at[pl.ds(ebase + c * CHUNK, CHUNK)],
                             src_v.at[q], isems[q])
            pltpu.async_copy(dst_hbm.at[pl.ds(ebase + c * CHUNK, CHUNK)],
                             dst_v.at[q], isems[q])

        def wait_idx(c, q):
            pltpu.make_async_copy(
                src_hbm.at[pl.ds(ebase + c * CHUNK, CHUNK)],
                src_v.at[q], isems[q]).wait()
            pltpu.make_async_copy(
                dst_hbm.at[pl.ds(ebase + c * CHUNK, CHUNK)],
                dst_v.at[q], isems[q]).wait()

        def start_chunk(c, p, q):
            pltpu.async_copy(h_hbm.at[src_v.at[q]], rows_v.at[p], gsems[p])
            pltpu.async_copy(he_hbm.at[pl.ds(ebase + c * CHUNK, CHUNK)],
                             he_v.at[p], hsems[p])

        def wait_chunk(c, p, q):
            pltpu.make_async_copy(
                h_hbm.at[src_v.at[q]], rows_v.at[p], gsems[p]).wait()
            pltpu.make_async_copy(
                he_hbm.at[pl.ds(ebase + c * CHUNK, CHUNK)],
                he_v.at[p], hsems[p]).wait()

        def compute(c, p):
            rows = rows_v.at[p]
            hes = he_v.at[p]

            @plsc.parallel_loop(0, CHUNK, unroll=4)
            def _(r):
                rrow = rows.at[r]
                herow = hes.at[r]
                for v in range(D // LANES):
                    sl = pl.ds(v * LANES, LANES)
                    rrow[sl] = jnp.maximum(rrow[sl] + herow[sl], 0.0)

        def start_scatter(c, p, q):
            pltpu.async_copy(rows_v.at[p], agg_sh.at[dst_v.at[q]], ssems[p],
                             add=True)

        def wait_scatter(c, p, q):
            pltpu.make_async_copy(
                rows_v.at[p], agg_sh.at[dst_v.at[q]], ssems[p]).wait()

        start_idx(0, 0)
        wait_idx(0, 0)
        start_chunk(0, 0, 0)
        start_idx(1, 1)
        start_idx(2, 2)

        @pl.loop(0, nch, step=4)
        def _(j):
            for u in range(4):
                c = j + u
                p = u % 2
                q = u % 4
                pn = (u + 1) % 2
                qn = (u + 1) % 4
                qf = (u + 3) % 4

                @pl.when(c + 1 < nch)
                def _():
                    wait_idx(c + 1, qn)

                @pl.when(c >= 1)
                def _():
                    wait_scatter(c - 1, pn, qf)

                @pl.when(c + 1 < nch)
                def _():
                    start_chunk(c + 1, pn, qn)

                @pl.when(c + 3 < nch)
                def _():
                    start_idx(c + 3, qf)

                wait_chunk(c, p, q)
                compute(c, p)
                start_scatter(c, p, q)

        wait_scatter(nch - 1, 1, 3)
        plsc.subcore_barrier()
        pltpu.sync_copy(agg_sh.at[pl.ds(sid * RPS, RPS)],
                        out_hbm.at[cid, pl.ds(sid * RPS, RPS)])

    return k(h, src2, dst2, he)


def kernel(x, edge_index, edge_attr, batch, mol_desc, W_in, b_in,
           W1_0, b1_0, W2_0, b2_0, g_0, be_0,
           W1_1, b1_1, W2_1, b2_1, g_1, be_1,
           W1_2, b1_2, W2_2, b2_2, g_2, be_2,
           W_fc, b_fc):
    pad = EPAD - E
    src2 = jnp.pad(edge_index[0], (0, pad))
    dst2 = jnp.pad(edge_index[1], (0, pad), constant_values=N)
    ea = jnp.pad(edge_attr, ((0, pad), (0, 0)))
    batch2d = batch.reshape(1, N)

    layers = [(W1_0, b1_0, W2_0, b2_0, g_0, be_0),
              (W1_1, b1_1, W2_1, b2_1, g_1, be_1),
              (W1_2, b1_2, W2_2, b2_2, g_2, be_2)]

    h = _inproj(x, W_in, b_in)
    hes = [_edge_mlp(ea, W1, b1, W2, b2)
           for (W1, b1, W2, b2, _, _) in layers]
    for l in range(L):
        agg = _sc_messages(h, src2, dst2, hes[l])
        h = _bn_update(h, agg, layers[l][4], layers[l][5])

    wh = W_fc[:D, 0].reshape(1, D)
    wm = W_fc[D:, 0].reshape(1, MD)
    out = _pool_fc(h, batch2d, mol_desc, wh, wm, b_fc)
    return out[:, 0]


# speedup vs baseline: 1.0745x; 1.0340x over previous
import functools

import jax
import jax.numpy as jnp
from jax import lax
from jax.experimental import pallas as pl
from jax.experimental.pallas import tpu as pltpu
from jax.experimental.pallas import tpu_sc as plsc

N = 10000
E = 320000
D = 128
ED = 16
G = 64
MD = 17
L = 3

NC = 2
NS = 16
NW = NC * NS
CHUNK = 64
NCH_A = 288
NCH_B = 32
EPS_A = NCH_A * CHUNK
EPS_B = NCH_B * CHUNK
EPAD = NS * (EPS_A + EPS_B)
NPAD = 10240
RPS = NPAD // NS
EB = 2048
LANES = 16


def _inproj(x, W, b):
    def body(x_ref, w_ref, b_ref, o_ref):
        o_ref[...] = jnp.dot(x_ref[...], w_ref[...],
                             preferred_element_type=jnp.float32) + b_ref[...]
    return pl.pallas_call(
        body, out_shape=jax.ShapeDtypeStruct((N, D), jnp.float32),
    )(x, W, b.reshape(1, D))


def _edge_mlp(ea, W1, b1, W2, b2):
    def body(ea_ref, w1_ref, b1_ref, w2_ref, b2_ref, o_ref):
        t = jnp.maximum(
            jnp.dot(ea_ref[...], w1_ref[...],
                    preferred_element_type=jnp.float32) + b1_ref[...], 0.0)
        o_ref[...] = jnp.dot(t, w2_ref[...],
                             preferred_element_type=jnp.float32) + b2_ref[...]
    return pl.pallas_call(
        body,
        grid=(EPAD // EB,),
        in_specs=[
            pl.BlockSpec((EB, ED), lambda i: (i, 0)),
            pl.BlockSpec((ED, D), lambda i: (0, 0)),
            pl.BlockSpec((1, D), lambda i: (0, 0)),
            pl.BlockSpec((D, D), lambda i: (0, 0)),
            pl.BlockSpec((1, D), lambda i: (0, 0)),
        ],
        out_specs=pl.BlockSpec((EB, D), lambda i: (i, 0)),
        out_shape=jax.ShapeDtypeStruct((EPAD, D), jnp.float32),
    )(ea, W1, b1.reshape(1, D), W2, b2.reshape(1, D))


def _bn_update(h, agg, g, be):
    def body(h_ref, a_ref, g_ref, be_ref, o_ref):
        z = h_ref[...] + a_ref[0] + a_ref[1]
        mean = jnp.mean(z, axis=0, keepdims=True)
        zc = z - mean
        var = jnp.mean(zc * zc, axis=0, keepdims=True)
        y = g_ref[...] * zc * lax.rsqrt(var + 1e-5) + be_ref[...]
        o_ref[...] = jnp.maximum(y, 0.0)
    return pl.pallas_call(
        body,
        grid=(1,),
        in_specs=[
            pl.BlockSpec((N, D), lambda i: (0, 0)),
            pl.BlockSpec((NC, N, D), lambda i: (0, 0, 0)),
            pl.BlockSpec((1, D), lambda i: (0, 0)),
            pl.BlockSpec((1, D), lambda i: (0, 0)),
        ],
        out_specs=pl.BlockSpec((N, D), lambda i: (0, 0)),
        out_shape=jax.ShapeDtypeStruct((N, D), jnp.float32),
    )(h, agg, g.reshape(1, D), be.reshape(1, D))


def _pool_fc(h, batch2d, mol_desc, wh, wm, bf):
    def body(h_ref, b_ref, md_ref, wh_ref, wm_ref, bf_ref, o_ref):
        gids = lax.broadcasted_iota(jnp.int32, (G, N), 0)
        M = (b_ref[...] == gids).astype(jnp.float32)
        sums = jnp.dot(M, h_ref[...], preferred_element_type=jnp.float32)
        counts = jnp.sum(M, axis=1, keepdims=True)
        pooled = sums / jnp.maximum(counts, 1.0)
        o_ref[...] = (jnp.sum(pooled * wh_ref[...], axis=1, keepdims=True)
                      + jnp.sum(md_ref[...] * wm_ref[...], axis=1, keepdims=True)
                      + bf_ref[0, 0])
    return pl.pallas_call(
        body, out_shape=jax.ShapeDtypeStruct((G, 1), jnp.float32),
    )(h, batch2d, mol_desc, wh, wm, bf.reshape(1, 1))


def _sc_messages(h, src2, dst2, he):
    mesh = plsc.VectorSubcoreMesh(core_axis_name="c", subcore_axis_name="s",
                                  num_cores=2)

    @functools.partial(
        pl.kernel,
        out_type=jax.ShapeDtypeStruct((NC, NPAD, D), jnp.float32),
        mesh=mesh,
        scratch_types=[
            pltpu.VMEM((4, CHUNK), jnp.int32),
            pltpu.VMEM((4, CHUNK), jnp.int32),
            pltpu.VMEM((2, CHUNK, D), jnp.float32),
            pltpu.VMEM((2, CHUNK, D), jnp.float32),
            pltpu.VMEM_SHARED((NPAD, D), jnp.float32),
            pltpu.SemaphoreType.DMA,
            pltpu.SemaphoreType.DMA,
            pltpu.SemaphoreType.DMA,
            pltpu.SemaphoreType.DMA,
            pltpu.SemaphoreType.DMA,
            pltpu.SemaphoreType.DMA,
            pltpu.SemaphoreType.DMA,
            pltpu.SemaphoreType.DMA,
            pltpu.SemaphoreType.DMA,
            pltpu.SemaphoreType.DMA,
        ],
    )
    def k(h_hbm, src_hbm, dst_hbm, he_hbm, out_hbm,
          src_v, dst_v, rows_v, he_v, agg_sh,
          is0, is1, is2, is3, gs0, gs1, hs0, hs1, ss0, ss1):
        cid = lax.axis_index("c")
        sid = lax.axis_index("s")
        on_a = cid == 0
        nch = jnp.where(on_a, NCH_A, NCH_B)
        ebase = jnp.where(on_a, sid * EPS_A, NS * EPS_A + sid * EPS_B)
        isems = (is0, is1, is2, is3)
        gsems = (gs0, gs1)
        hsems = (hs0, hs1)
        ssems = (ss0, ss1)

        zbuf = rows_v.at[0]

        @pl.loop(0, CHUNK)
        def _(r):
            zrow = zbuf.at[r]
            for v in range(D // LANES):
                zrow[pl.ds(v * LANES, LANES)] = jnp.zeros((LANES,),
                                                          jnp.float32)

        @pl.loop(0, RPS // CHUNK)
        def _(t):
            pltpu.sync_copy(
                zbuf, agg_sh.at[pl.ds(sid * RPS + t * CHUNK, CHUNK)])
        plsc.subcore_barrier()

        def start_idx(c, q):
            pltpu.async_copy(src_hbm.at[pl.ds(ebase + c * CHUNK, CHUNK)],
                             src_v.at[q], isems[q])
            pltpu.async_copy(dst_hbm.at[pl.ds(ebase + c * CHUNK, CHUNK)],
                             dst_v.at[q], isems[q])

        def wait_idx(c, q):
            pltpu.make_async_copy(
                src_hbm.at[pl.ds(ebase + c * CHUNK, CHUNK)],
                src_v.at[q], isems[q]).wait()
            pltpu.make_async_copy(
                dst_hbm.at[pl.ds(ebase + c * CHUNK, CHUNK)],
                dst_v.at[q], isems[q]).wait()

        def start_chunk(c, p, q):
            pltpu.async_copy(h_hbm.at[src_v.at[q]], rows_v.at[p], gsems[p])
            pltpu.async_copy(he_hbm.at[pl.ds(ebase + c * CHUNK, CHUNK)],
                             he_v.at[p], hsems[p])

        def wait_chunk(c, p, q):
            pltpu.make_async_copy(
                h_hbm.at[src_v.at[q]], rows_v.at[p], gsems[p]).wait()
            pltpu.make_async_copy(
                he_hbm.at[pl.ds(ebase + c * CHUNK, CHUNK)],
                he_v.at[p], hsems[p]).wait()

        def compute(c, p):
            rows = rows_v.at[p]
            hes = he_v.at[p]

            @plsc.parallel_loop(0, CHUNK, unroll=4)
            def _(r):
                rrow = rows.at[r]
                herow = hes.at[r]
                for v in range(D // LANES):
                    sl = pl.ds(v * LANES, LANES)
                    rrow[sl] = jnp.maximum(rrow[sl] + herow[sl], 0.0)

        def start_scatter(c, p, q):
            pltpu.async_copy(rows_v.at[p], agg_sh.at[dst_v.at[q]], ssems[p],
                             add=True)

        def wait_scatter(c, p, q):
            pltpu.make_async_copy(
                rows_v.at[p], agg_sh.at[dst_v.at[q]], ssems[p]).wait()

        start_idx(0, 0)
        wait_idx(0, 0)
        start_chunk(0, 0, 0)
        start_idx(1, 1)
        start_idx(2, 2)

        @pl.loop(0, nch, step=4)
        def _(j):
            for u in range(4):
                c = j + u
                p = u % 2
                q = u % 4
                pn = (u + 1) % 2
                qn = (u + 1) % 4
                qf = (u + 3) % 4

                @pl.when(c + 1 < nch)
                def _():
                    wait_idx(c + 1, qn)

                @pl.when(c >= 1)
                def _():
                    wait_scatter(c - 1, pn, qf)

                @pl.when(c + 1 < nch)
                def _():
                    start_chunk(c + 1, pn, qn)

                @pl.when(c + 3 < nch)
                def _():
                    start_idx(c + 3, qf)

                wait_chunk(c, p, q)
                compute(c, p)
                start_scatter(c, p, q)

        wait_scatter(nch - 1, 1, 3)
        plsc.subcore_barrier()
        pltpu.sync_copy(agg_sh.at[pl.ds(sid * RPS, RPS)],
                        out_hbm.at[cid, pl.ds(sid * RPS, RPS)])

    return k(h, src2, dst2, he)


def kernel(x, edge_index, edge_attr, batch, mol_desc, W_in, b_in,
           W1_0, b1_0, W2_0, b2_0, g_0, be_0,
           W1_1, b1_1, W2_1, b2_1, g_1, be_1,
           W1_2, b1_2, W2_2, b2_2, g_2, be_2,
           W_fc, b_fc):
    pad = EPAD - E
    src2 = jnp.pad(edge_index[0], (0, pad))
    dst2 = jnp.pad(edge_index[1], (0, pad), constant_values=N)
    ea = jnp.pad(edge_attr, ((0, pad), (0, 0)))
    batch2d = batch.reshape(1, N)

    layers = [(W1_0, b1_0, W2_0, b2_0, g_0, be_0),
              (W1_1, b1_1, W2_1, b2_1, g_1, be_1),
              (W1_2, b1_2, W2_2, b2_2, g_2, be_2)]

    h = _inproj(x, W_in, b_in)
    hes = [_edge_mlp(ea, W1, b1, W2, b2)
           for (W1, b1, W2, b2, _, _) in layers]
    for l in range(L):
        agg = _sc_messages(h, src2, dst2, hes[l])
        h = _bn_update(h, agg, layers[l][4], layers[l][5])

    wh = W_fc[:D, 0].reshape(1, D)
    wm = W_fc[D:, 0].reshape(1, MD)
    out = _pool_fc(h, batch2d, mol_desc, wh, wm, b_fc)
    return out[:, 0]
